# Initial kernel scaffold; baseline (speedup 1.0000x reference)
import jax
import jax.numpy as jnp
from jax.experimental import pallas as pl


def kernel(x, Ws, bs, Wr, br, Wg, bg):
    raise NotImplementedError("write your pallas kernel here")


# speedup vs baseline: 2.5994x; 2.5994x over previous
import functools

import jax
import jax.numpy as jnp
from jax.experimental import pallas as pl
from jax.experimental.pallas import tpu as pltpu

_BLK = 256


def _moe_body(x_ref, Ws_ref, bs_ref, Wr_ref, br_ref, Wg_ref, bg_ref,
              out_ref, logits_ref, wsum_bf, wr_bf, bsum_ref):
    j = pl.program_id(0)
    E = Wr_ref.shape[0]
    H = x_ref.shape[1]

    @pl.when(j == 0)
    def _prep():
        ws = Ws_ref[0]
        for e in range(1, E):
            ws = ws + Ws_ref[e]
        wsum_bf[...] = ws.astype(jnp.bfloat16)
        wr_bf[...] = Wr_ref[...].astype(jnp.bfloat16)
        bsum_ref[...] = jnp.sum(bs_ref[...], axis=0, keepdims=True)

    xb = x_ref[...]

    logits = jax.lax.dot_general(
        xb.astype(jnp.bfloat16), Wg_ref[...].astype(jnp.bfloat16),
        (((1,), (0,)), ((), ())),
        preferred_element_type=jnp.float32) + bg_ref[...]
    logits_ref[...] = logits

    iota = jax.lax.broadcasted_iota(jnp.int32, logits.shape, 1)
    m1 = jnp.max(logits, axis=1, keepdims=True)
    a1 = jnp.min(jnp.where(logits == m1, iota, E), axis=1, keepdims=True)
    masked = jnp.where(iota == a1, -jnp.inf, logits)
    m2 = jnp.max(masked, axis=1, keepdims=True)
    a2 = jnp.min(jnp.where(masked == m2, iota, E), axis=1, keepdims=True)
    w1 = 1.0 / (1.0 + jnp.exp(m2 - m1))
    w2 = 1.0 - w1
    comb = (jnp.where(iota == a1, w1, 0.0)
            + jnp.where(iota == a2, w2, 0.0))

    xh = xb.astype(jnp.bfloat16)
    dot = functools.partial(
        jax.lax.dot_general,
        dimension_numbers=(((1,), (0,)), ((), ())),
        preferred_element_type=jnp.float32)
    acc = dot(xh, wsum_bf[...])
    for e in range(E):
        ye = dot(xh, wr_bf[e])
        acc = acc + ye * comb[:, e:e + 1]
    rbias = jax.lax.dot_general(
        comb, br_ref[...], (((1,), (0,)), ((), ())),
        precision=jax.lax.Precision.HIGHEST,
        preferred_element_type=jnp.float32)
    out_ref[...] = acc + bsum_ref[...] + rbias


def kernel(x, Ws, bs, Wr, br, Wg, bg):
    b, s, h = x.shape
    E = Ws.shape[0]
    T = b * s
    x2 = x.reshape(T, h)
    bg2 = bg.reshape(1, E)
    nblk = T // _BLK

    out, logits = pl.pallas_call(
        _moe_body,
        grid=(nblk,),
        in_specs=[
            pl.BlockSpec((_BLK, h), lambda j: (j, 0)),
            pl.BlockSpec((E, h, h), lambda j: (0, 0, 0)),
            pl.BlockSpec((E, h), lambda j: (0, 0)),
            pl.BlockSpec((E, h, h), lambda j: (0, 0, 0)),
            pl.BlockSpec((E, h), lambda j: (0, 0)),
            pl.BlockSpec((h, E), lambda j: (0, 0)),
            pl.BlockSpec((1, E), lambda j: (0, 0)),
        ],
        out_specs=[
            pl.BlockSpec((_BLK, h), lambda j: (j, 0)),
            pl.BlockSpec((_BLK, E), lambda j: (j, 0)),
        ],
        out_shape=[
            jax.ShapeDtypeStruct((T, h), jnp.float32),
            jax.ShapeDtypeStruct((T, E), jnp.float32),
        ],
        scratch_shapes=[
            pltpu.VMEM((h, h), jnp.bfloat16),
            pltpu.VMEM((E, h, h), jnp.bfloat16),
            pltpu.VMEM((1, h), jnp.float32),
        ],
    )(x2, Ws, bs, Wr, br, Wg, bg2)

    return out.reshape(b, s, h), logits


# speedup vs baseline: 2.6310x; 1.0122x over previous
import functools

import jax
import jax.numpy as jnp
from jax.experimental import pallas as pl
from jax.experimental.pallas import tpu as pltpu

_BLK = 256


def _moe_body(x_ref, Ws_ref, bs_ref, Wr_ref, br_ref, Wg_ref, bg_ref,
              out_ref, logits_ref, wsum_bf, bsum_ref):
    j = pl.program_id(0)
    E = Wr_ref.shape[0]
    H = x_ref.shape[1]

    @pl.when(j == 0)
    def _prep():
        ws = Ws_ref[0]
        for e in range(1, E):
            ws = ws + Ws_ref[e]
        wsum_bf[...] = ws
        bsum_ref[...] = jnp.sum(bs_ref[...], axis=0, keepdims=True)

    xb = x_ref[...]

    logits = jax.lax.dot_general(
        xb.astype(jnp.bfloat16), Wg_ref[...].astype(jnp.bfloat16),
        (((1,), (0,)), ((), ())),
        preferred_element_type=jnp.float32) + bg_ref[...]
    logits_ref[...] = logits

    iota = jax.lax.broadcasted_iota(jnp.int32, logits.shape, 1)
    m1 = jnp.max(logits, axis=1, keepdims=True)
    a1 = jnp.min(jnp.where(logits == m1, iota, E), axis=1, keepdims=True)
    masked = jnp.where(iota == a1, -jnp.inf, logits)
    m2 = jnp.max(masked, axis=1, keepdims=True)
    a2 = jnp.min(jnp.where(masked == m2, iota, E), axis=1, keepdims=True)
    w1 = 1.0 / (1.0 + jnp.exp(m2 - m1))
    w2 = 1.0 - w1
    comb = (jnp.where(iota == a1, w1, 0.0)
            + jnp.where(iota == a2, w2, 0.0))

    dot = functools.partial(
        jax.lax.dot_general,
        dimension_numbers=(((1,), (0,)), ((), ())),
        preferred_element_type=jnp.float32)
    acc = dot(xb, wsum_bf[...])
    for e in range(E):
        ye = dot(xb, Wr_ref[e])
        acc = acc + ye * comb[:, e:e + 1]
    rbias = jax.lax.dot_general(
        comb, br_ref[...], (((1,), (0,)), ((), ())),
        precision=jax.lax.Precision.HIGHEST,
        preferred_element_type=jnp.float32)
    out_ref[...] = acc + bsum_ref[...] + rbias


def kernel(x, Ws, bs, Wr, br, Wg, bg):
    b, s, h = x.shape
    E = Ws.shape[0]
    T = b * s
    x2 = x.reshape(T, h)
    bg2 = bg.reshape(1, E)
    nblk = T // _BLK

    out, logits = pl.pallas_call(
        _moe_body,
        grid=(nblk,),
        in_specs=[
            pl.BlockSpec((_BLK, h), lambda j: (j, 0)),
            pl.BlockSpec((E, h, h), lambda j: (0, 0, 0)),
            pl.BlockSpec((E, h), lambda j: (0, 0)),
            pl.BlockSpec((E, h, h), lambda j: (0, 0, 0)),
            pl.BlockSpec((E, h), lambda j: (0, 0)),
            pl.BlockSpec((h, E), lambda j: (0, 0)),
            pl.BlockSpec((1, E), lambda j: (0, 0)),
        ],
        out_specs=[
            pl.BlockSpec((_BLK, h), lambda j: (j, 0)),
            pl.BlockSpec((_BLK, E), lambda j: (j, 0)),
        ],
        out_shape=[
            jax.ShapeDtypeStruct((T, h), jnp.float32),
            jax.ShapeDtypeStruct((T, E), jnp.float32),
        ],
        scratch_shapes=[
            pltpu.VMEM((h, h), jnp.float32),
            pltpu.VMEM((1, h), jnp.float32),
        ],
    )(x2, Ws, bs, Wr, br, Wg, bg2)

    return out.reshape(b, s, h), logits


# speedup vs baseline: 2.7823x; 1.0575x over previous
import functools

import jax
import jax.numpy as jnp
from jax.experimental import pallas as pl
from jax.experimental.pallas import tpu as pltpu


def _moe_body(x_ref, Ws_ref, bs_ref, Wr_ref, br_ref, Wg_ref, bg_ref,
              out_ref, logits_ref, xbf_ref, acc_ref, accw_ref, comb_ref):
    e = pl.program_id(0)
    E = pl.num_programs(0) - 1

    dot = functools.partial(
        jax.lax.dot_general,
        dimension_numbers=(((1,), (0,)), ((), ())),
        preferred_element_type=jnp.float32)

    @pl.when(e == 0)
    def _router():
        xb = x_ref[...]
        xbf_ref[...] = xb.astype(jnp.bfloat16)
        logits = dot(xbf_ref[...], Wg_ref[...].astype(jnp.bfloat16)) + bg_ref[...]
        logits_ref[...] = logits
        iota = jax.lax.broadcasted_iota(jnp.int32, logits.shape, 1)
        m1 = jnp.max(logits, axis=1, keepdims=True)
        a1 = jnp.min(jnp.where(logits == m1, iota, E), axis=1, keepdims=True)
        masked = jnp.where(iota == a1, -jnp.inf, logits)
        m2 = jnp.max(masked, axis=1, keepdims=True)
        a2 = jnp.min(jnp.where(masked == m2, iota, E), axis=1, keepdims=True)
        w1 = 1.0 / (1.0 + jnp.exp(m2 - m1))
        w2 = 1.0 - w1
        comb_ref[...] = (jnp.where(iota == a1, w1, 0.0)
                         + jnp.where(iota == a2, w2, 0.0))
        accw_ref[...] = Ws_ref[0]

    @pl.when((e > 0) & (e < E))
    def _accw():
        accw_ref[...] += Ws_ref[0]

    @pl.when(e < E)
    def _expert():
        ye = dot(xbf_ref[...], Wr_ref[0].astype(jnp.bfloat16))
        comb = comb_ref[...]
        lane = jax.lax.broadcasted_iota(jnp.int32, comb.shape, 1)
        c_col = jnp.sum(jnp.where(lane == e, comb, 0.0), axis=1,
                        keepdims=True)
        ye = ye * c_col

        @pl.when(e == 0)
        def _():
            acc_ref[...] = ye

        @pl.when(e > 0)
        def _():
            acc_ref[...] += ye

    @pl.when(e == E)
    def _final():
        shared = dot(xbf_ref[...], accw_ref[...].astype(jnp.bfloat16))
        bsum = jnp.sum(bs_ref[...], axis=0, keepdims=True)
        rbias = dot(comb_ref[...].astype(jnp.bfloat16),
                    br_ref[...].astype(jnp.bfloat16))
        out_ref[...] = acc_ref[...] + shared + bsum + rbias


def kernel(x, Ws, bs, Wr, br, Wg, bg):
    b, s, h = x.shape
    E = Ws.shape[0]
    T = b * s
    x2 = x.reshape(T, h)
    bg2 = bg.reshape(1, E)

    def wblk(e):
        return (jnp.minimum(e, E - 1), 0, 0)

    out, logits = pl.pallas_call(
        _moe_body,
        grid=(E + 1,),
        in_specs=[
            pl.BlockSpec((T, h), lambda e: (0, 0)),
            pl.BlockSpec((1, h, h), lambda e: (jnp.minimum(e, E - 1), 0, 0)),
            pl.BlockSpec((E, h), lambda e: (0, 0)),
            pl.BlockSpec((1, h, h), lambda e: (jnp.minimum(e, E - 1), 0, 0)),
            pl.BlockSpec((E, h), lambda e: (0, 0)),
            pl.BlockSpec((h, E), lambda e: (0, 0)),
            pl.BlockSpec((1, E), lambda e: (0, 0)),
        ],
        out_specs=[
            pl.BlockSpec((T, h), lambda e: (0, 0)),
            pl.BlockSpec((T, E), lambda e: (0, 0)),
        ],
        out_shape=[
            jax.ShapeDtypeStruct((T, h), jnp.float32),
            jax.ShapeDtypeStruct((T, E), jnp.float32),
        ],
        scratch_shapes=[
            pltpu.VMEM((T, h), jnp.bfloat16),
            pltpu.VMEM((T, h), jnp.float32),
            pltpu.VMEM((h, h), jnp.float32),
            pltpu.VMEM((T, E), jnp.float32),
        ],
    )(x2, Ws, bs, Wr, br, Wg, bg2)

    return out.reshape(b, s, h), logits


# speedup vs baseline: 2.9902x; 1.0747x over previous
import functools

import jax
import jax.numpy as jnp
from jax.experimental import pallas as pl
from jax.experimental.pallas import tpu as pltpu

_PAIR = 2


def _moe_body(x_ref, Ws_ref, bs_ref, Wr_ref, br_ref, Wg_ref, bg_ref,
              out_ref, logits_ref, xbf_ref, acc_ref, accw_ref, comb_ref):
    p = pl.program_id(0)
    NP = pl.num_programs(0) - 1

    dot = functools.partial(
        jax.lax.dot_general,
        dimension_numbers=(((1,), (0,)), ((), ())),
        preferred_element_type=jnp.float32)

    @pl.when(p == 0)
    def _router():
        xbf_ref[...] = x_ref[...].astype(jnp.bfloat16)
        logits = dot(xbf_ref[...], Wg_ref[...].astype(jnp.bfloat16)) + bg_ref[...]
        logits_ref[...] = logits
        iota = jax.lax.broadcasted_iota(jnp.int32, logits.shape, 1)
        E = logits.shape[1]
        m1 = jnp.max(logits, axis=1, keepdims=True)
        a1 = jnp.min(jnp.where(logits == m1, iota, E), axis=1, keepdims=True)
        masked = jnp.where(iota == a1, -jnp.inf, logits)
        m2 = jnp.max(masked, axis=1, keepdims=True)
        a2 = jnp.min(jnp.where(masked == m2, iota, E), axis=1, keepdims=True)
        w1 = 1.0 / (1.0 + jnp.exp(m2 - m1))
        w2 = 1.0 - w1
        comb_ref[...] = (jnp.where(iota == a1, w1, 0.0)
                         + jnp.where(iota == a2, w2, 0.0))
        accw_ref[...] = Ws_ref[0] + Ws_ref[1]

    @pl.when((p > 0) & (p < NP))
    def _accw():
        accw_ref[...] += Ws_ref[0] + Ws_ref[1]

    @pl.when(p < NP)
    def _experts():
        comb = comb_ref[...]
        lane = jax.lax.broadcasted_iota(jnp.int32, comb.shape, 1)
        part = None
        for k in range(_PAIR):
            ye = dot(xbf_ref[...], Wr_ref[k].astype(jnp.bfloat16))
            e = p * _PAIR + k
            c_col = jnp.sum(jnp.where(lane == e, comb, 0.0), axis=1,
                            keepdims=True)
            yw = (ye * c_col).astype(jnp.bfloat16)
            part = yw if part is None else part + yw

        @pl.when(p == 0)
        def _():
            acc_ref[...] = part

        @pl.when(p > 0)
        def _():
            acc_ref[...] += part

    @pl.when(p == NP)
    def _final():
        shared = dot(xbf_ref[...], accw_ref[...].astype(jnp.bfloat16))
        bsum = jnp.sum(bs_ref[...], axis=0, keepdims=True)
        rbias = dot(comb_ref[...].astype(jnp.bfloat16),
                    br_ref[...].astype(jnp.bfloat16))
        out_ref[...] = acc_ref[...].astype(jnp.float32) + shared + bsum + rbias


def kernel(x, Ws, bs, Wr, br, Wg, bg):
    b, s, h = x.shape
    E = Ws.shape[0]
    T = b * s
    x2 = x.reshape(T, h)
    bg2 = bg.reshape(1, E)
    npair = E // _PAIR

    def wmap(p):
        return (jnp.minimum(p, npair - 1), 0, 0)

    out, logits = pl.pallas_call(
        _moe_body,
        grid=(npair + 1,),
        in_specs=[
            pl.BlockSpec((T, h), lambda p: (0, 0)),
            pl.BlockSpec((_PAIR, h, h), wmap),
            pl.BlockSpec((E, h), lambda p: (0, 0)),
            pl.BlockSpec((_PAIR, h, h), wmap),
            pl.BlockSpec((E, h), lambda p: (0, 0)),
            pl.BlockSpec((h, E), lambda p: (0, 0)),
            pl.BlockSpec((1, E), lambda p: (0, 0)),
        ],
        out_specs=[
            pl.BlockSpec((T, h), lambda p: (0, 0)),
            pl.BlockSpec((T, E), lambda p: (0, 0)),
        ],
        out_shape=[
            jax.ShapeDtypeStruct((T, h), jnp.float32),
            jax.ShapeDtypeStruct((T, E), jnp.float32),
        ],
        scratch_shapes=[
            pltpu.VMEM((T, h), jnp.bfloat16),
            pltpu.VMEM((T, h), jnp.bfloat16),
            pltpu.VMEM((h, h), jnp.float32),
            pltpu.VMEM((T, E), jnp.float32),
        ],
    )(x2, Ws, bs, Wr, br, Wg, bg2)

    return out.reshape(b, s, h), logits


# speedup vs baseline: 3.0061x; 1.0053x over previous
import functools

import jax
import jax.numpy as jnp
from jax.experimental import pallas as pl
from jax.experimental.pallas import tpu as pltpu

_PAIR = 2


def _moe_body(x_ref, Ws_ref, bs_ref, Wr_ref, br_ref, Wg_ref, bg_ref,
              out_ref, logits_ref, xbf_ref, acc_ref, accw_ref, comb_ref):
    p = pl.program_id(0)
    NP = pl.num_programs(0)

    dot = functools.partial(
        jax.lax.dot_general,
        dimension_numbers=(((1,), (0,)), ((), ())),
        preferred_element_type=jnp.float32)

    @pl.when(p == 0)
    def _router():
        xbf_ref[...] = x_ref[...].astype(jnp.bfloat16)
        logits = dot(xbf_ref[...], Wg_ref[...].astype(jnp.bfloat16)) + bg_ref[...]
        logits_ref[...] = logits
        iota = jax.lax.broadcasted_iota(jnp.int32, logits.shape, 1)
        E = logits.shape[1]
        m1 = jnp.max(logits, axis=1, keepdims=True)
        a1 = jnp.min(jnp.where(logits == m1, iota, E), axis=1, keepdims=True)
        masked = jnp.where(iota == a1, -jnp.inf, logits)
        m2 = jnp.max(masked, axis=1, keepdims=True)
        a2 = jnp.min(jnp.where(masked == m2, iota, E), axis=1, keepdims=True)
        w1 = 1.0 / (1.0 + jnp.exp(m2 - m1))
        w2 = 1.0 - w1
        comb_ref[...] = (jnp.where(iota == a1, w1, 0.0)
                         + jnp.where(iota == a2, w2, 0.0))
        accw_ref[...] = Ws_ref[0] + Ws_ref[1]

    @pl.when(p > 0)
    def _accw():
        accw_ref[...] += Ws_ref[0] + Ws_ref[1]

    comb = comb_ref[...]
    lane = jax.lax.broadcasted_iota(jnp.int32, comb.shape, 1)
    part = None
    for k in range(_PAIR):
        ye = dot(xbf_ref[...], Wr_ref[k].astype(jnp.bfloat16))
        e = p * _PAIR + k
        c_col = jnp.sum(jnp.where(lane == e, comb, 0.0), axis=1,
                        keepdims=True)
        yw = (ye * c_col).astype(jnp.bfloat16)
        part = yw if part is None else part + yw

    @pl.when(p == 0)
    def _():
        acc_ref[...] = part

    @pl.when((p > 0) & (p < NP - 1))
    def _():
        acc_ref[...] += part

    @pl.when(p == NP - 1)
    def _final():
        shared = dot(xbf_ref[...], accw_ref[...].astype(jnp.bfloat16))
        bsum = jnp.sum(bs_ref[...], axis=0, keepdims=True)
        rbias = dot(comb_ref[...].astype(jnp.bfloat16),
                    br_ref[...].astype(jnp.bfloat16))
        out_ref[...] = ((acc_ref[...] + part).astype(jnp.float32)
                        + shared + bsum + rbias)


def kernel(x, Ws, bs, Wr, br, Wg, bg):
    b, s, h = x.shape
    E = Ws.shape[0]
    T = b * s
    x2 = x.reshape(T, h)
    bg2 = bg.reshape(1, E)
    npair = E // _PAIR

    def wmap(p):
        return (p, 0, 0)

    out, logits = pl.pallas_call(
        _moe_body,
        grid=(npair,),
        in_specs=[
            pl.BlockSpec((T, h), lambda p: (0, 0)),
            pl.BlockSpec((_PAIR, h, h), wmap),
            pl.BlockSpec((E, h), lambda p: (0, 0)),
            pl.BlockSpec((_PAIR, h, h), wmap),
            pl.BlockSpec((E, h), lambda p: (0, 0)),
            pl.BlockSpec((h, E), lambda p: (0, 0)),
            pl.BlockSpec((1, E), lambda p: (0, 0)),
        ],
        out_specs=[
            pl.BlockSpec((T, h), lambda p: (0, 0)),
            pl.BlockSpec((T, E), lambda p: (0, 0)),
        ],
        out_shape=[
            jax.ShapeDtypeStruct((T, h), jnp.float32),
            jax.ShapeDtypeStruct((T, E), jnp.float32),
        ],
        scratch_shapes=[
            pltpu.VMEM((T, h), jnp.bfloat16),
            pltpu.VMEM((T, h), jnp.bfloat16),
            pltpu.VMEM((h, h), jnp.float32),
            pltpu.VMEM((T, E), jnp.float32),
        ],
    )(x2, Ws, bs, Wr, br, Wg, bg2)

    return out.reshape(b, s, h), logits
